# Initial kernel scaffold; baseline (speedup 1.0000x reference)
import jax
import jax.numpy as jnp
from jax.experimental import pallas as pl


def kernel(x0, x1, edge_index0, edge_index1, Win0, bin0, eWl0, ebl0, eWr0, ebr0, ea0, eb0, dWl0, dbl0, dWr0, dbr0, da0, db0, Wout0, bout0, Win1, bin1, eWl1, ebl1, eWr1, ebr1, ea1, eb1, dWl1, dbl1, dWr1, dbr1, da1, db1, Wout1, bout1):
    raise NotImplementedError("write your pallas kernel here")


# speedup vs baseline: 6.2918x; 6.2918x over previous
import functools

import jax
import jax.numpy as jnp
from jax import lax
from jax.experimental import pallas as pl
from jax.experimental.pallas import tpu as pltpu
from jax.experimental.pallas import tpu_sc as plsc

_N = 10000
_E = 320000
_D_IN = 128
_D_HID = 256
_D_Z = 128

_NC = 2
_NS = 16
_NW = _NC * _NS
_EPW = _E // _NW
_NPAD = 10240
_RPT = _NPAD // _NS


def _make_edge_sc(d_logit, separate_agg, bsz):
    ndl = d_logit // 16
    nchunk = _EPW // bsz
    mesh = plsc.VectorSubcoreMesh(core_axis_name="c", subcore_axis_name="s")

    scratch = [
        pltpu.VMEM_SHARED((_NPAD, 128), jnp.float32),
        pltpu.VMEM_SHARED((_NPAD,), jnp.float32),
        pltpu.VMEM((bsz,), jnp.int32),
        pltpu.VMEM((bsz,), jnp.int32),
        pltpu.VMEM((bsz, d_logit), jnp.float32),
        pltpu.VMEM((bsz, d_logit), jnp.float32),
        pltpu.VMEM((d_logit,), jnp.float32),
        pltpu.VMEM((bsz,), jnp.float32),
        pltpu.VMEM((256,), jnp.float32),
        pltpu.VMEM((16, 128), jnp.float32),
        pltpu.VMEM((_RPT,), jnp.float32),
        pltpu.SemaphoreType.DMA,
        pltpu.SemaphoreType.DMA,
    ]
    if separate_agg:
        scratch.insert(6, pltpu.VMEM((bsz, 128), jnp.float32))
        scratch.append(pltpu.SemaphoreType.DMA)

    def body(*refs):
        if separate_agg:
            (src_hbm, dst_hbm, xl_hbm, xr_hbm, ag_hbm, att_hbm,
             acc_hbm, den_hbm,
             acc_sh, den_sh, sidx, didx, xl_v, xr_v, ag_v, att_v, w_v,
             tr_v, zr_v, zd_v, sem1, sem2, sem3) = refs
        else:
            (src_hbm, dst_hbm, xl_hbm, xr_hbm, att_hbm,
             acc_hbm, den_hbm,
             acc_sh, den_sh, sidx, didx, xl_v, xr_v, att_v, w_v,
             tr_v, zr_v, zd_v, sem1, sem2) = refs
            ag_v = xl_v
        cid = lax.axis_index("c")
        sid = lax.axis_index("s")
        wid = cid * _NS + sid

        def _zr(i, _):
            for c in range(8):
                zr_v[i, pl.ds(c * 16, 16)] = jnp.zeros((16,), jnp.float32)
            return 0
        lax.fori_loop(0, 16, _zr, 0)

        def _zd(i, _):
            zd_v[pl.ds(i * 16, 16)] = jnp.zeros((16,), jnp.float32)
            return 0
        lax.fori_loop(0, _RPT // 16, _zd, 0)

        base_r = sid * _RPT

        def _zacc(k, _):
            pltpu.sync_copy(zr_v, acc_sh.at[pl.ds(base_r + k * 16, 16)])
            return 0
        lax.fori_loop(0, _RPT // 16, _zacc, 0)
        pltpu.sync_copy(zd_v, den_sh.at[pl.ds(base_r, _RPT)])
        pltpu.sync_copy(att_hbm, att_v)
        plsc.subcore_barrier()

        def chunk(ci, _):
            eb = wid * _EPW + ci * bsz
            pltpu.sync_copy(src_hbm.at[pl.ds(eb, bsz)], sidx)
            pltpu.sync_copy(dst_hbm.at[pl.ds(eb, bsz)], didx)
            cp1 = pltpu.async_copy(xl_hbm.at[sidx], xl_v, sem1)
            cp2 = pltpu.async_copy(xr_hbm.at[didx], xr_v, sem2)
            if separate_agg:
                cp3 = pltpu.async_copy(ag_hbm.at[sidx], ag_v, sem3)
            cp1.wait()
            cp2.wait()
            if separate_agg:
                cp3.wait()

            lane = lax.broadcasted_iota(jnp.int32, (16,), 0)
            tr_idx = [lane * 16 + l for l in range(16)]
            splat_idx = [jnp.full((16,), l, jnp.int32) for l in range(16)]

            def group(g, _):
                for l in range(16):
                    j = g * 16 + l
                    acc = jnp.zeros((16,), jnp.float32)
                    for d in range(ndl):
                        a = xl_v[j, pl.ds(d * 16, 16)]
                        b = xr_v[j, pl.ds(d * 16, 16)]
                        s = a + b
                        s = jnp.maximum(s, s * 0.2)
                        acc = acc + s * att_v[pl.ds(d * 16, 16)]
                    plsc.store_scatter(tr_v, [tr_idx[l]], acc)
                t16 = tr_v[pl.ds(0, 16)]
                for l in range(1, 16):
                    t16 = t16 + tr_v[pl.ds(l * 16, 16)]
                w16 = jnp.exp(t16)
                w_v[pl.ds(g * 16, 16)] = w16
                for l in range(16):
                    j = g * 16 + l
                    wl = jnp.take_along_axis(
                        w16, splat_idx[l], axis=0, mode="promise_in_bounds")
                    for d in range(8):
                        ag_v[j, pl.ds(d * 16, 16)] = (
                            ag_v[j, pl.ds(d * 16, 16)] * wl)
                return 0
            lax.fori_loop(0, bsz // 16, group, 0)

            pltpu.sync_copy(ag_v, acc_sh.at[didx], add=True)
            pltpu.sync_copy(w_v, den_sh.at[didx], add=True)
            return 0
        lax.fori_loop(0, nchunk, chunk, 0)
        plsc.subcore_barrier()

        out_r = cid * _NPAD + base_r

        def _out(k, _):
            pltpu.sync_copy(acc_sh.at[pl.ds(base_r + k * 16, 16)], zr_v)
            pltpu.sync_copy(zr_v, acc_hbm.at[pl.ds(out_r + k * 16, 16)])
            return 0
        lax.fori_loop(0, _RPT // 16, _out, 0)
        pltpu.sync_copy(den_sh.at[pl.ds(base_r, _RPT)], zd_v)
        pltpu.sync_copy(zd_v, den_hbm.at[pl.ds(out_r, _RPT)])

    return pl.kernel(
        body,
        out_type=[
            jax.ShapeDtypeStruct((_NC * _NPAD, 128), jnp.float32),
            jax.ShapeDtypeStruct((_NC * _NPAD,), jnp.float32),
        ],
        mesh=mesh,
        scratch_types=scratch,
        compiler_params=pltpu.CompilerParams(needs_layout_passes=False),
    )


_enc_edge = _make_edge_sc(_D_Z, separate_agg=False, bsz=80)
_dec_edge = _make_edge_sc(_D_HID, separate_agg=True, bsz=16)


_BR = 1000
_GRID = _N // _BR


def _row_spec(cols):
    return pl.BlockSpec((_BR, cols), lambda i: (i, 0))


def _full2(r, c):
    return pl.BlockSpec((r, c), lambda i: (0, 0))


def _full1(n):
    return pl.BlockSpec((n,), lambda i: (0,))


def _acc_spec():
    return pl.BlockSpec((2, _BR, 128), lambda i: (0, i, 0))


def _den_spec():
    return pl.BlockSpec((2, _BR, 1), lambda i: (0, i, 0))


def _lrelu(v):
    return jnp.maximum(v, v * 0.2)


def _tc_a_body(x0, Win0, bin0, eWl0, ebl0, eWr0, ebr0,
               x1, Win1, bin1, eWl1, ebl1, eWr1, ebr1,
               exl0, exr0, exl1, exr1):
    f32 = jnp.float32
    h0 = jnp.dot(x0[...], Win0[...], preferred_element_type=f32) + bin0[...]
    h1 = jnp.dot(x1[...], Win1[...], preferred_element_type=f32) + bin1[...]
    exl0[...] = jnp.dot(h0, eWl0[...], preferred_element_type=f32) + ebl0[...]
    exr0[...] = jnp.dot(h0, eWr0[...], preferred_element_type=f32) + ebr0[...]
    exl1[...] = jnp.dot(h1, eWl1[...], preferred_element_type=f32) + ebl1[...]
    exr1[...] = jnp.dot(h1, eWr1[...], preferred_element_type=f32) + ebr1[...]


def _tc_a(x0, Win0, bin0, eWl0, ebl0, eWr0, ebr0,
          x1, Win1, bin1, eWl1, ebl1, eWr1, ebr1):
    o = jax.ShapeDtypeStruct((_N, _D_Z), jnp.float32)
    return pl.pallas_call(
        _tc_a_body,
        grid=(_GRID,),
        in_specs=[
            _row_spec(_D_IN), _full2(_D_IN, _D_HID), _full1(_D_HID),
            _full2(_D_HID, _D_Z), _full1(_D_Z),
            _full2(_D_HID, _D_Z), _full1(_D_Z),
            _row_spec(_D_IN), _full2(_D_IN, _D_HID), _full1(_D_HID),
            _full2(_D_HID, _D_Z), _full1(_D_Z),
            _full2(_D_HID, _D_Z), _full1(_D_Z),
        ],
        out_specs=[_row_spec(_D_Z)] * 4,
        out_shape=[o, o, o, o],
    )(x0, Win0, bin0, eWl0, ebl0, eWr0, ebr0,
      x1, Win1, bin1, eWl1, ebl1, eWr1, ebr1)


def _tc_b_body(acc0, den0, acc1, den1, exl0, exr0, exl1, exr1,
               ea0, eb0, ea1, eb1,
               dWl0, dbl0, dWr0, dbr0, dWl1, dbl1, dWr1, dbr1,
               z0o, z1o, zo, dxl0, dxr0, dxl1, dxr1):
    f32 = jnp.float32
    a0 = acc0[0] + acc0[1]
    d0 = den0[0] + den0[1]
    a1 = acc1[0] + acc1[1]
    d1 = den1[0] + den1[1]
    xl0, xr0 = exl0[...], exr0[...]
    xl1, xr1 = exl1[...], exr1[...]
    w0 = jnp.exp(jnp.dot(_lrelu(xl0 + xr0), ea0[...], preferred_element_type=f32))
    w1 = jnp.exp(jnp.dot(_lrelu(xl1 + xr1), ea1[...], preferred_element_type=f32))
    z0 = (a0 + w0 * xl0) / (d0 + w0) + eb0[...]
    z1 = (a1 + w1 * xl1) / (d1 + w1) + eb1[...]
    z = z0 + z1
    z0o[...] = z0
    z1o[...] = z1
    zo[...] = z
    dxl0[...] = jnp.dot(z, dWl0[...], preferred_element_type=f32) + dbl0[...]
    dxr0[...] = jnp.dot(z, dWr0[...], preferred_element_type=f32) + dbr0[...]
    dxl1[...] = jnp.dot(z, dWl1[...], preferred_element_type=f32) + dbl1[...]
    dxr1[...] = jnp.dot(z, dWr1[...], preferred_element_type=f32) + dbr1[...]


def _tc_b(acc0, den0, acc1, den1, exl0, exr0, exl1, exr1,
          ea0, eb0, ea1, eb1,
          dWl0, dbl0, dWr0, dbr0, dWl1, dbl1, dWr1, dbr1):
    oz = jax.ShapeDtypeStruct((_N, _D_Z), jnp.float32)
    oh = jax.ShapeDtypeStruct((_N, _D_HID), jnp.float32)
    return pl.pallas_call(
        _tc_b_body,
        grid=(_GRID,),
        in_specs=[
            _acc_spec(), _den_spec(), _acc_spec(), _den_spec(),
            _row_spec(_D_Z), _row_spec(_D_Z), _row_spec(_D_Z), _row_spec(_D_Z),
            _full2(_D_Z, 1), _full1(_D_Z), _full2(_D_Z, 1), _full1(_D_Z),
            _full2(_D_Z, _D_HID), _full1(_D_HID),
            _full2(_D_Z, _D_HID), _full1(_D_HID),
            _full2(_D_Z, _D_HID), _full1(_D_HID),
            _full2(_D_Z, _D_HID), _full1(_D_HID),
        ],
        out_specs=[_row_spec(_D_Z)] * 3 + [_row_spec(_D_HID)] * 4,
        out_shape=[oz, oz, oz, oh, oh, oh, oh],
    )(acc0, den0, acc1, den1, exl0, exr0, exl1, exr1,
      ea0, eb0, ea1, eb1,
      dWl0, dbl0, dWr0, dbr0, dWl1, dbl1, dWr1, dbr1)


def _tc_c_body(accz0, denz0, accz1, denz1, dxl0, dxr0, dxl1, dxr1, z,
               da0, da1, dWl0, dbl0, db0, dWl1, dbl1, db1,
               Wout0, bout0, Wout1, bout1,
               xh0, xh1):
    f32 = jnp.float32
    zb = z[...]
    a0 = accz0[0] + accz0[1]
    d0 = denz0[0] + denz0[1]
    a1 = accz1[0] + accz1[1]
    d1 = denz1[0] + denz1[1]
    w0 = jnp.exp(jnp.dot(_lrelu(dxl0[...] + dxr0[...]), da0[...],
                         preferred_element_type=f32))
    w1 = jnp.exp(jnp.dot(_lrelu(dxl1[...] + dxr1[...]), da1[...],
                         preferred_element_type=f32))
    s0 = (a0 + w0 * zb) / (d0 + w0)
    s1 = (a1 + w1 * zb) / (d1 + w1)
    hd0 = jnp.dot(s0, dWl0[...], preferred_element_type=f32) + dbl0[...] + db0[...]
    hd1 = jnp.dot(s1, dWl1[...], preferred_element_type=f32) + dbl1[...] + db1[...]
    xh0[...] = jnp.dot(hd0, Wout0[...], preferred_element_type=f32) + bout0[...]
    xh1[...] = jnp.dot(hd1, Wout1[...], preferred_element_type=f32) + bout1[...]


def _tc_c(accz0, denz0, accz1, denz1, dxl0, dxr0, dxl1, dxr1, z,
          da0, da1, dWl0, dbl0, db0, dWl1, dbl1, db1,
          Wout0, bout0, Wout1, bout1):
    o = jax.ShapeDtypeStruct((_N, _D_IN), jnp.float32)
    return pl.pallas_call(
        _tc_c_body,
        grid=(_GRID,),
        in_specs=[
            _acc_spec(), _den_spec(), _acc_spec(), _den_spec(),
            _row_spec(_D_HID), _row_spec(_D_HID),
            _row_spec(_D_HID), _row_spec(_D_HID),
            _row_spec(_D_Z),
            _full2(_D_HID, 1), _full2(_D_HID, 1),
            _full2(_D_Z, _D_HID), _full1(_D_HID), _full1(_D_HID),
            _full2(_D_Z, _D_HID), _full1(_D_HID), _full1(_D_HID),
            _full2(_D_HID, _D_IN), _full1(_D_IN),
            _full2(_D_HID, _D_IN), _full1(_D_IN),
        ],
        out_specs=[_row_spec(_D_IN)] * 2,
        out_shape=[o, o],
    )(accz0, denz0, accz1, denz1, dxl0, dxr0, dxl1, dxr1, z,
      da0, da1, dWl0, dbl0, db0, dWl1, dbl1, db1,
      Wout0, bout0, Wout1, bout1)


def _split_acc(acc_flat, den_flat):
    acc = acc_flat.reshape(_NC, _NPAD, 128)[:, :_N]
    den = den_flat.reshape(_NC, _NPAD)[:, :_N, None]
    return acc, den


def kernel(x0, x1, edge_index0, edge_index1,
           Win0, bin0, eWl0, ebl0, eWr0, ebr0, ea0, eb0,
           dWl0, dbl0, dWr0, dbr0, da0, db0, Wout0, bout0,
           Win1, bin1, eWl1, ebl1, eWr1, ebr1, ea1, eb1,
           dWl1, dbl1, dWr1, dbr1, da1, db1, Wout1, bout1):
    src0, dst0 = edge_index0[0], edge_index0[1]
    src1, dst1 = edge_index1[0], edge_index1[1]

    exl0, exr0, exl1, exr1 = _tc_a(
        x0, Win0, bin0, eWl0, ebl0, eWr0, ebr0,
        x1, Win1, bin1, eWl1, ebl1, eWr1, ebr1)

    accf0, denf0 = _enc_edge(src0, dst0, exl0, exr0, ea0)
    accf1, denf1 = _enc_edge(src1, dst1, exl1, exr1, ea1)
    acc0, den0 = _split_acc(accf0, denf0)
    acc1, den1 = _split_acc(accf1, denf1)

    z0, z1, z, dxl0, dxr0, dxl1, dxr1 = _tc_b(
        acc0, den0, acc1, den1, exl0, exr0, exl1, exr1,
        ea0.reshape(_D_Z, 1), eb0, ea1.reshape(_D_Z, 1), eb1,
        dWl0, dbl0, dWr0, dbr0, dWl1, dbl1, dWr1, dbr1)

    aczf0, dezf0 = _dec_edge(src1, dst1, dxl0, dxr0, z, da0)
    aczf1, dezf1 = _dec_edge(src1, dst1, dxl1, dxr1, z, da1)
    accz0, denz0 = _split_acc(aczf0, dezf0)
    accz1, denz1 = _split_acc(aczf1, dezf1)

    xh0, xh1 = _tc_c(
        accz0, denz0, accz1, denz1, dxl0, dxr0, dxl1, dxr1, z,
        da0.reshape(_D_HID, 1), da1.reshape(_D_HID, 1),
        dWl0, dbl0, db0, dWl1, dbl1, db1,
        Wout0, bout0, Wout1, bout1)

    return (xh0, xh1, z0, z1, z)


# speedup vs baseline: 9.3226x; 1.4817x over previous
import functools

import jax
import jax.numpy as jnp
from jax import lax
from jax.experimental import pallas as pl
from jax.experimental.pallas import tpu as pltpu
from jax.experimental.pallas import tpu_sc as plsc

_N = 10000
_E = 320000
_D_IN = 128
_D_HID = 256
_D_Z = 128

_NC = 2
_NS = 16
_NW = _NC * _NS
_EPW = _E // _NW
_NPAD = 10240
_RPT = _NPAD // _NS


def _make_edge_sc(d_logit, separate_agg, bsz, logit_bf16=False):
    ndl = d_logit // 16
    nchunk = _EPW // bsz
    lshape = (bsz, d_logit // 2) if logit_bf16 else (bsz, d_logit)
    ldt = jnp.int32 if logit_bf16 else jnp.float32
    mesh = plsc.VectorSubcoreMesh(core_axis_name="c", subcore_axis_name="s")

    scratch = [
        pltpu.VMEM_SHARED((_NPAD, 128), jnp.float32),
        pltpu.VMEM_SHARED((_NPAD,), jnp.float32),
        pltpu.VMEM((bsz,), jnp.int32),
        pltpu.VMEM((bsz,), jnp.int32),
        pltpu.VMEM(lshape, ldt),
        pltpu.VMEM(lshape, ldt),
        pltpu.VMEM((d_logit,), jnp.float32),
        pltpu.VMEM((bsz,), jnp.float32),
        pltpu.VMEM((256,), jnp.float32),
        pltpu.VMEM((16, 128), jnp.float32),
        pltpu.VMEM((_RPT,), jnp.float32),
        pltpu.SemaphoreType.DMA,
        pltpu.SemaphoreType.DMA,
    ]
    if separate_agg:
        scratch.insert(6, pltpu.VMEM((bsz, 128), jnp.float32))
        scratch.append(pltpu.SemaphoreType.DMA)

    def body(*refs):
        if separate_agg:
            (src_hbm, dst_hbm, xl_hbm, xr_hbm, ag_hbm, att_hbm,
             acc_hbm, den_hbm,
             acc_sh, den_sh, sidx, didx, xl_v, xr_v, ag_v, att_v, w_v,
             tr_v, zr_v, zd_v, sem1, sem2, sem3) = refs
        else:
            (src_hbm, dst_hbm, xl_hbm, xr_hbm, att_hbm,
             acc_hbm, den_hbm,
             acc_sh, den_sh, sidx, didx, xl_v, xr_v, att_v, w_v,
             tr_v, zr_v, zd_v, sem1, sem2) = refs
            ag_v = xl_v
        cid = lax.axis_index("c")
        sid = lax.axis_index("s")
        wid = cid * _NS + sid

        def _zr(i, _):
            for c in range(8):
                zr_v[i, pl.ds(c * 16, 16)] = jnp.zeros((16,), jnp.float32)
            return 0
        lax.fori_loop(0, 16, _zr, 0)

        def _zd(i, _):
            zd_v[pl.ds(i * 16, 16)] = jnp.zeros((16,), jnp.float32)
            return 0
        lax.fori_loop(0, _RPT // 16, _zd, 0)

        base_r = sid * _RPT

        def _zacc(k, _):
            pltpu.sync_copy(zr_v, acc_sh.at[pl.ds(base_r + k * 16, 16)])
            return 0
        lax.fori_loop(0, _RPT // 16, _zacc, 0)
        pltpu.sync_copy(zd_v, den_sh.at[pl.ds(base_r, _RPT)])
        pltpu.sync_copy(att_hbm, att_v)
        plsc.subcore_barrier()

        def chunk(ci, _):
            eb = wid * _EPW + ci * bsz
            pltpu.sync_copy(src_hbm.at[pl.ds(eb, bsz)], sidx)
            pltpu.sync_copy(dst_hbm.at[pl.ds(eb, bsz)], didx)
            cp1 = pltpu.async_copy(xl_hbm.at[sidx], xl_v, sem1)
            cp2 = pltpu.async_copy(xr_hbm.at[didx], xr_v, sem2)
            if separate_agg:
                cp3 = pltpu.async_copy(ag_hbm.at[sidx], ag_v, sem3)
            cp1.wait()
            cp2.wait()
            if separate_agg:
                cp3.wait()

            lane = lax.broadcasted_iota(jnp.int32, (16,), 0)
            tr_idx = [lane * 16 + l for l in range(16)]
            splat_idx = [jnp.full((16,), l, jnp.int32) for l in range(16)]

            def group(g, _):
                for l in range(16):
                    j = g * 16 + l
                    acc = jnp.zeros((16,), jnp.float32)
                    if logit_bf16:
                        for d in range(d_logit // 32):
                            a2 = plsc.bitcast(
                                xl_v[j, pl.ds(d * 16, 16)], jnp.bfloat16)
                            b2 = plsc.bitcast(
                                xr_v[j, pl.ds(d * 16, 16)], jnp.bfloat16)
                            ae, ao = plsc.unpack(
                                a2, format=plsc.PackFormat.INTERLEAVED)
                            be, bo = plsc.unpack(
                                b2, format=plsc.PackFormat.INTERLEAVED)
                            se = ae + be
                            se = jnp.maximum(se, se * 0.2)
                            so = ao + bo
                            so = jnp.maximum(so, so * 0.2)
                            acc = acc + se * att_v[pl.ds(d * 32, 16)]
                            acc = acc + so * att_v[pl.ds(d * 32 + 16, 16)]
                    else:
                        for d in range(ndl):
                            a = xl_v[j, pl.ds(d * 16, 16)]
                            b = xr_v[j, pl.ds(d * 16, 16)]
                            s = a + b
                            s = jnp.maximum(s, s * 0.2)
                            acc = acc + s * att_v[pl.ds(d * 16, 16)]
                    plsc.store_scatter(tr_v, [tr_idx[l]], acc)
                t16 = tr_v[pl.ds(0, 16)]
                for l in range(1, 16):
                    t16 = t16 + tr_v[pl.ds(l * 16, 16)]
                w16 = jnp.exp(t16)
                w_v[pl.ds(g * 16, 16)] = w16
                for l in range(16):
                    j = g * 16 + l
                    wl = jnp.take_along_axis(
                        w16, splat_idx[l], axis=0, mode="promise_in_bounds")
                    for d in range(8):
                        ag_v[j, pl.ds(d * 16, 16)] = (
                            ag_v[j, pl.ds(d * 16, 16)] * wl)
                return 0
            lax.fori_loop(0, bsz // 16, group, 0)

            pltpu.sync_copy(ag_v, acc_sh.at[didx], add=True)
            pltpu.sync_copy(w_v, den_sh.at[didx], add=True)
            return 0
        lax.fori_loop(0, nchunk, chunk, 0)
        plsc.subcore_barrier()

        out_r = cid * _NPAD + base_r

        def _out(k, _):
            pltpu.sync_copy(acc_sh.at[pl.ds(base_r + k * 16, 16)], zr_v)
            pltpu.sync_copy(zr_v, acc_hbm.at[pl.ds(out_r + k * 16, 16)])
            return 0
        lax.fori_loop(0, _RPT // 16, _out, 0)
        pltpu.sync_copy(den_sh.at[pl.ds(base_r, _RPT)], zd_v)
        pltpu.sync_copy(zd_v, den_hbm.at[pl.ds(out_r, _RPT)])

    return pl.kernel(
        body,
        out_type=[
            jax.ShapeDtypeStruct((_NC * _NPAD, 128), jnp.float32),
            jax.ShapeDtypeStruct((_NC * _NPAD,), jnp.float32),
        ],
        mesh=mesh,
        scratch_types=scratch,
        compiler_params=pltpu.CompilerParams(needs_layout_passes=False),
    )


_enc_edge = _make_edge_sc(_D_Z, separate_agg=False, bsz=80)
_dec_edge = _make_edge_sc(_D_HID, separate_agg=True, bsz=80, logit_bf16=True)


def _perm_even_odd(att):
    d = att.shape[0]
    return att.reshape(d // 32, 16, 2).swapaxes(1, 2).reshape(d)


def _as_i32_pairs(x):
    n, d = x.shape
    return lax.bitcast_convert_type(
        x.astype(jnp.bfloat16).reshape(n, d // 2, 2), jnp.int32)


_BR = 1000
_GRID = _N // _BR


def _row_spec(cols):
    return pl.BlockSpec((_BR, cols), lambda i: (i, 0))


def _full2(r, c):
    return pl.BlockSpec((r, c), lambda i: (0, 0))


def _full1(n):
    return pl.BlockSpec((n,), lambda i: (0,))


def _acc_spec():
    return pl.BlockSpec((2, _BR, 128), lambda i: (0, i, 0))


def _den_spec():
    return pl.BlockSpec((2, _BR, 1), lambda i: (0, i, 0))


def _lrelu(v):
    return jnp.maximum(v, v * 0.2)


def _tc_a_body(x0, Win0, bin0, eWl0, ebl0, eWr0, ebr0,
               x1, Win1, bin1, eWl1, ebl1, eWr1, ebr1,
               exl0, exr0, exl1, exr1):
    f32 = jnp.float32
    h0 = jnp.dot(x0[...], Win0[...], preferred_element_type=f32) + bin0[...]
    h1 = jnp.dot(x1[...], Win1[...], preferred_element_type=f32) + bin1[...]
    exl0[...] = jnp.dot(h0, eWl0[...], preferred_element_type=f32) + ebl0[...]
    exr0[...] = jnp.dot(h0, eWr0[...], preferred_element_type=f32) + ebr0[...]
    exl1[...] = jnp.dot(h1, eWl1[...], preferred_element_type=f32) + ebl1[...]
    exr1[...] = jnp.dot(h1, eWr1[...], preferred_element_type=f32) + ebr1[...]


def _tc_a(x0, Win0, bin0, eWl0, ebl0, eWr0, ebr0,
          x1, Win1, bin1, eWl1, ebl1, eWr1, ebr1):
    o = jax.ShapeDtypeStruct((_N, _D_Z), jnp.float32)
    return pl.pallas_call(
        _tc_a_body,
        grid=(_GRID,),
        in_specs=[
            _row_spec(_D_IN), _full2(_D_IN, _D_HID), _full1(_D_HID),
            _full2(_D_HID, _D_Z), _full1(_D_Z),
            _full2(_D_HID, _D_Z), _full1(_D_Z),
            _row_spec(_D_IN), _full2(_D_IN, _D_HID), _full1(_D_HID),
            _full2(_D_HID, _D_Z), _full1(_D_Z),
            _full2(_D_HID, _D_Z), _full1(_D_Z),
        ],
        out_specs=[_row_spec(_D_Z)] * 4,
        out_shape=[o, o, o, o],
    )(x0, Win0, bin0, eWl0, ebl0, eWr0, ebr0,
      x1, Win1, bin1, eWl1, ebl1, eWr1, ebr1)


def _tc_b_body(acc0, den0, acc1, den1, exl0, exr0, exl1, exr1,
               ea0, eb0, ea1, eb1,
               dWl0, dbl0, dWr0, dbr0, dWl1, dbl1, dWr1, dbr1,
               z0o, z1o, zo, dxl0, dxr0, dxl1, dxr1):
    f32 = jnp.float32
    a0 = acc0[0] + acc0[1]
    d0 = den0[0] + den0[1]
    a1 = acc1[0] + acc1[1]
    d1 = den1[0] + den1[1]
    xl0, xr0 = exl0[...], exr0[...]
    xl1, xr1 = exl1[...], exr1[...]
    w0 = jnp.exp(jnp.dot(_lrelu(xl0 + xr0), ea0[...], preferred_element_type=f32))
    w1 = jnp.exp(jnp.dot(_lrelu(xl1 + xr1), ea1[...], preferred_element_type=f32))
    z0 = (a0 + w0 * xl0) / (d0 + w0) + eb0[...]
    z1 = (a1 + w1 * xl1) / (d1 + w1) + eb1[...]
    z = z0 + z1
    z0o[...] = z0
    z1o[...] = z1
    zo[...] = z
    dxl0[...] = jnp.dot(z, dWl0[...], preferred_element_type=f32) + dbl0[...]
    dxr0[...] = jnp.dot(z, dWr0[...], preferred_element_type=f32) + dbr0[...]
    dxl1[...] = jnp.dot(z, dWl1[...], preferred_element_type=f32) + dbl1[...]
    dxr1[...] = jnp.dot(z, dWr1[...], preferred_element_type=f32) + dbr1[...]


def _tc_b(acc0, den0, acc1, den1, exl0, exr0, exl1, exr1,
          ea0, eb0, ea1, eb1,
          dWl0, dbl0, dWr0, dbr0, dWl1, dbl1, dWr1, dbr1):
    oz = jax.ShapeDtypeStruct((_N, _D_Z), jnp.float32)
    oh = jax.ShapeDtypeStruct((_N, _D_HID), jnp.float32)
    return pl.pallas_call(
        _tc_b_body,
        grid=(_GRID,),
        in_specs=[
            _acc_spec(), _den_spec(), _acc_spec(), _den_spec(),
            _row_spec(_D_Z), _row_spec(_D_Z), _row_spec(_D_Z), _row_spec(_D_Z),
            _full2(_D_Z, 1), _full1(_D_Z), _full2(_D_Z, 1), _full1(_D_Z),
            _full2(_D_Z, _D_HID), _full1(_D_HID),
            _full2(_D_Z, _D_HID), _full1(_D_HID),
            _full2(_D_Z, _D_HID), _full1(_D_HID),
            _full2(_D_Z, _D_HID), _full1(_D_HID),
        ],
        out_specs=[_row_spec(_D_Z)] * 3 + [_row_spec(_D_HID)] * 4,
        out_shape=[oz, oz, oz, oh, oh, oh, oh],
    )(acc0, den0, acc1, den1, exl0, exr0, exl1, exr1,
      ea0, eb0, ea1, eb1,
      dWl0, dbl0, dWr0, dbr0, dWl1, dbl1, dWr1, dbr1)


def _tc_c_body(accz0, denz0, accz1, denz1, dxl0, dxr0, dxl1, dxr1, z,
               da0, da1, dWl0, dbl0, db0, dWl1, dbl1, db1,
               Wout0, bout0, Wout1, bout1,
               xh0, xh1):
    f32 = jnp.float32
    zb = z[...]
    a0 = accz0[0] + accz0[1]
    d0 = denz0[0] + denz0[1]
    a1 = accz1[0] + accz1[1]
    d1 = denz1[0] + denz1[1]
    w0 = jnp.exp(jnp.dot(_lrelu(dxl0[...] + dxr0[...]), da0[...],
                         preferred_element_type=f32))
    w1 = jnp.exp(jnp.dot(_lrelu(dxl1[...] + dxr1[...]), da1[...],
                         preferred_element_type=f32))
    s0 = (a0 + w0 * zb) / (d0 + w0)
    s1 = (a1 + w1 * zb) / (d1 + w1)
    hd0 = jnp.dot(s0, dWl0[...], preferred_element_type=f32) + dbl0[...] + db0[...]
    hd1 = jnp.dot(s1, dWl1[...], preferred_element_type=f32) + dbl1[...] + db1[...]
    xh0[...] = jnp.dot(hd0, Wout0[...], preferred_element_type=f32) + bout0[...]
    xh1[...] = jnp.dot(hd1, Wout1[...], preferred_element_type=f32) + bout1[...]


def _tc_c(accz0, denz0, accz1, denz1, dxl0, dxr0, dxl1, dxr1, z,
          da0, da1, dWl0, dbl0, db0, dWl1, dbl1, db1,
          Wout0, bout0, Wout1, bout1):
    o = jax.ShapeDtypeStruct((_N, _D_IN), jnp.float32)
    return pl.pallas_call(
        _tc_c_body,
        grid=(_GRID,),
        in_specs=[
            _acc_spec(), _den_spec(), _acc_spec(), _den_spec(),
            _row_spec(_D_HID), _row_spec(_D_HID),
            _row_spec(_D_HID), _row_spec(_D_HID),
            _row_spec(_D_Z),
            _full2(_D_HID, 1), _full2(_D_HID, 1),
            _full2(_D_Z, _D_HID), _full1(_D_HID), _full1(_D_HID),
            _full2(_D_Z, _D_HID), _full1(_D_HID), _full1(_D_HID),
            _full2(_D_HID, _D_IN), _full1(_D_IN),
            _full2(_D_HID, _D_IN), _full1(_D_IN),
        ],
        out_specs=[_row_spec(_D_IN)] * 2,
        out_shape=[o, o],
    )(accz0, denz0, accz1, denz1, dxl0, dxr0, dxl1, dxr1, z,
      da0, da1, dWl0, dbl0, db0, dWl1, dbl1, db1,
      Wout0, bout0, Wout1, bout1)


def _split_acc(acc_flat, den_flat):
    acc = acc_flat.reshape(_NC, _NPAD, 128)[:, :_N]
    den = den_flat.reshape(_NC, _NPAD)[:, :_N, None]
    return acc, den


def kernel(x0, x1, edge_index0, edge_index1,
           Win0, bin0, eWl0, ebl0, eWr0, ebr0, ea0, eb0,
           dWl0, dbl0, dWr0, dbr0, da0, db0, Wout0, bout0,
           Win1, bin1, eWl1, ebl1, eWr1, ebr1, ea1, eb1,
           dWl1, dbl1, dWr1, dbr1, da1, db1, Wout1, bout1):
    src0, dst0 = edge_index0[0], edge_index0[1]
    src1, dst1 = edge_index1[0], edge_index1[1]

    exl0, exr0, exl1, exr1 = _tc_a(
        x0, Win0, bin0, eWl0, ebl0, eWr0, ebr0,
        x1, Win1, bin1, eWl1, ebl1, eWr1, ebr1)

    accf0, denf0 = _enc_edge(src0, dst0, exl0, exr0, ea0)
    accf1, denf1 = _enc_edge(src1, dst1, exl1, exr1, ea1)
    acc0, den0 = _split_acc(accf0, denf0)
    acc1, den1 = _split_acc(accf1, denf1)

    z0, z1, z, dxl0, dxr0, dxl1, dxr1 = _tc_b(
        acc0, den0, acc1, den1, exl0, exr0, exl1, exr1,
        ea0.reshape(_D_Z, 1), eb0, ea1.reshape(_D_Z, 1), eb1,
        dWl0, dbl0, dWr0, dbr0, dWl1, dbl1, dWr1, dbr1)

    aczf0, dezf0 = _dec_edge(src1, dst1, _as_i32_pairs(dxl0),
                             _as_i32_pairs(dxr0), z, _perm_even_odd(da0))
    aczf1, dezf1 = _dec_edge(src1, dst1, _as_i32_pairs(dxl1),
                             _as_i32_pairs(dxr1), z, _perm_even_odd(da1))
    accz0, denz0 = _split_acc(aczf0, dezf0)
    accz1, denz1 = _split_acc(aczf1, dezf1)

    xh0, xh1 = _tc_c(
        accz0, denz0, accz1, denz1, dxl0, dxr0, dxl1, dxr1, z,
        da0.reshape(_D_HID, 1), da1.reshape(_D_HID, 1),
        dWl0, dbl0, db0, dWl1, dbl1, db1,
        Wout0, bout0, Wout1, bout1)

    return (xh0, xh1, z0, z1, z)
